# baseline (device time: 11519 ns/iter reference)
import jax
import jax.numpy as jnp
from jax import lax
from jax.experimental import pallas as pl
from jax.experimental.pallas import tpu as pltpu

DY_ROWS = 320
N_FWD_CHUNKS = 2


def kernel(x):
    m, n = x.shape
    fwd_rows = m - DY_ROWS
    fc = fwd_rows // N_FWD_CHUNKS

    def body(x_ref, out_ref, ys_send, ys_recv, xs_send, xs_recv):
        my_x = lax.axis_index("x")
        my_y = lax.axis_index("y")
        y_peer = (my_x, 1 - my_y)
        x_peer = (1 - my_x, my_y)

        own_base = my_y * m
        opp_base = (1 - my_y) * m

        barrier_sem = pltpu.get_barrier_semaphore()
        for peer in (y_peer, x_peer):
            pl.semaphore_signal(
                barrier_sem, inc=1, device_id=peer,
                device_id_type=pl.DeviceIdType.MESH,
            )
        pl.semaphore_wait(barrier_sem, 2)

        keep_lo = jnp.where(my_x == 0, fwd_rows, m - DY_ROWS)
        keep_sz = DY_ROWS - fwd_rows
        send_offs = []
        for c in range(N_FWD_CHUNKS):
            lo = c * fc
            hi = DY_ROWS + c * fc
            send_offs.append(jnp.where(my_x == 0, lo, hi))

        y_sends = []
        for c in range(N_FWD_CHUNKS):
            off = send_offs[c]
            out_ref[pl.ds(own_base + off, fc), :] = x_ref[
                pl.ds(off, fc), :
            ].astype(out_ref.dtype)
            s = pltpu.make_async_remote_copy(
                src_ref=out_ref.at[pl.ds(own_base + off, fc), :],
                dst_ref=out_ref.at[pl.ds(own_base + off, fc), :],
                send_sem=ys_send.at[c],
                recv_sem=ys_recv.at[c],
                device_id=y_peer,
                device_id_type=pl.DeviceIdType.MESH,
            )
            s.start()
            y_sends.append(s)
        out_ref[pl.ds(own_base + keep_lo, keep_sz), :] = x_ref[
            pl.ds(keep_lo, keep_sz), :
        ].astype(out_ref.dtype)
        s = pltpu.make_async_remote_copy(
            src_ref=out_ref.at[pl.ds(own_base + keep_lo, keep_sz), :],
            dst_ref=out_ref.at[pl.ds(own_base + keep_lo, keep_sz), :],
            send_sem=ys_send.at[N_FWD_CHUNKS],
            recv_sem=ys_recv.at[N_FWD_CHUNKS],
            device_id=y_peer,
            device_id_type=pl.DeviceIdType.MESH,
        )
        s.start()
        y_sends.append(s)

        rest_lo = jnp.where(my_x == 0, DY_ROWS, 0)
        out_ref[pl.ds(own_base + rest_lo, fwd_rows), :] = x_ref[
            pl.ds(rest_lo, fwd_rows), :
        ].astype(out_ref.dtype)

        x_sends = []
        for c in range(N_FWD_CHUNKS):
            off = send_offs[c]
            rows = pl.ds(opp_base + off, fc)
            r = pltpu.make_async_remote_copy(
                src_ref=out_ref.at[rows, :],
                dst_ref=out_ref.at[rows, :],
                send_sem=ys_send.at[c],
                recv_sem=ys_recv.at[c],
                device_id=y_peer,
                device_id_type=pl.DeviceIdType.MESH,
            )
            r.wait_recv()
            f = pltpu.make_async_remote_copy(
                src_ref=out_ref.at[rows, :],
                dst_ref=out_ref.at[rows, :],
                send_sem=xs_send.at[c],
                recv_sem=xs_recv.at[c],
                device_id=x_peer,
                device_id_type=pl.DeviceIdType.MESH,
            )
            f.start()
            x_sends.append(f)

        keep = pltpu.make_async_remote_copy(
            src_ref=out_ref.at[pl.ds(opp_base + keep_lo, keep_sz), :],
            dst_ref=out_ref.at[pl.ds(opp_base + keep_lo, keep_sz), :],
            send_sem=ys_send.at[N_FWD_CHUNKS],
            recv_sem=ys_recv.at[N_FWD_CHUNKS],
            device_id=y_peer,
            device_id_type=pl.DeviceIdType.MESH,
        )
        keep.wait_recv()

        for c in range(N_FWD_CHUNKS):
            off = jnp.where(my_x == 0, DY_ROWS + c * fc, c * fc)
            rows = pl.ds(opp_base + off, fc)
            r = pltpu.make_async_remote_copy(
                src_ref=out_ref.at[rows, :],
                dst_ref=out_ref.at[rows, :],
                send_sem=xs_send.at[c],
                recv_sem=xs_recv.at[c],
                device_id=x_peer,
                device_id_type=pl.DeviceIdType.MESH,
            )
            r.wait_recv()

        for s in y_sends:
            s.wait_send()
        for s in x_sends:
            s.wait_send()

    return pl.pallas_call(
        body,
        out_shape=jax.ShapeDtypeStruct((2 * m, n), jnp.bfloat16),
        in_specs=[pl.BlockSpec(memory_space=pltpu.VMEM)],
        out_specs=pl.BlockSpec(memory_space=pltpu.VMEM),
        scratch_shapes=[
            pltpu.SemaphoreType.DMA((N_FWD_CHUNKS + 1,)),
            pltpu.SemaphoreType.DMA((N_FWD_CHUNKS + 1,)),
            pltpu.SemaphoreType.DMA((N_FWD_CHUNKS,)),
            pltpu.SemaphoreType.DMA((N_FWD_CHUNKS,)),
        ],
        compiler_params=pltpu.CompilerParams(collective_id=0),
    )(x)


# device time: 11360 ns/iter; 1.0140x vs baseline; 1.0140x over previous
import jax
import jax.numpy as jnp
from jax import lax
from jax.experimental import pallas as pl
from jax.experimental.pallas import tpu as pltpu

DY_ROWS = 384
N_FWD_CHUNKS = 2


def kernel(x):
    m, n = x.shape
    fwd_rows = m - DY_ROWS
    fc = fwd_rows // N_FWD_CHUNKS

    def body(x_ref, out_ref, ys_send, ys_recv, xs_send, xs_recv):
        my_x = lax.axis_index("x")
        my_y = lax.axis_index("y")
        y_peer = (my_x, 1 - my_y)
        x_peer = (1 - my_x, my_y)

        own_base = my_y * m
        opp_base = (1 - my_y) * m

        barrier_sem = pltpu.get_barrier_semaphore()
        for peer in (y_peer, x_peer):
            pl.semaphore_signal(
                barrier_sem, inc=1, device_id=peer,
                device_id_type=pl.DeviceIdType.MESH,
            )
        pl.semaphore_wait(barrier_sem, 2)

        keep_lo = jnp.where(my_x == 0, fwd_rows, m - DY_ROWS)
        keep_sz = DY_ROWS - fwd_rows
        send_offs = []
        for c in range(N_FWD_CHUNKS):
            lo = c * fc
            hi = DY_ROWS + c * fc
            send_offs.append(jnp.where(my_x == 0, lo, hi))

        y_sends = []
        for c in range(N_FWD_CHUNKS):
            off = send_offs[c]
            out_ref[pl.ds(own_base + off, fc), :] = x_ref[
                pl.ds(off, fc), :
            ].astype(out_ref.dtype)
            s = pltpu.make_async_remote_copy(
                src_ref=out_ref.at[pl.ds(own_base + off, fc), :],
                dst_ref=out_ref.at[pl.ds(own_base + off, fc), :],
                send_sem=ys_send.at[c],
                recv_sem=ys_recv.at[c],
                device_id=y_peer,
                device_id_type=pl.DeviceIdType.MESH,
            )
            s.start()
            y_sends.append(s)
        out_ref[pl.ds(own_base + keep_lo, keep_sz), :] = x_ref[
            pl.ds(keep_lo, keep_sz), :
        ].astype(out_ref.dtype)
        s = pltpu.make_async_remote_copy(
            src_ref=out_ref.at[pl.ds(own_base + keep_lo, keep_sz), :],
            dst_ref=out_ref.at[pl.ds(own_base + keep_lo, keep_sz), :],
            send_sem=ys_send.at[N_FWD_CHUNKS],
            recv_sem=ys_recv.at[N_FWD_CHUNKS],
            device_id=y_peer,
            device_id_type=pl.DeviceIdType.MESH,
        )
        s.start()
        y_sends.append(s)

        rest_lo = jnp.where(my_x == 0, DY_ROWS, 0)
        out_ref[pl.ds(own_base + rest_lo, fwd_rows), :] = x_ref[
            pl.ds(rest_lo, fwd_rows), :
        ].astype(out_ref.dtype)

        x_sends = []
        for c in range(N_FWD_CHUNKS):
            off = send_offs[c]
            rows = pl.ds(opp_base + off, fc)
            r = pltpu.make_async_remote_copy(
                src_ref=out_ref.at[rows, :],
                dst_ref=out_ref.at[rows, :],
                send_sem=ys_send.at[c],
                recv_sem=ys_recv.at[c],
                device_id=y_peer,
                device_id_type=pl.DeviceIdType.MESH,
            )
            r.wait_recv()
            f = pltpu.make_async_remote_copy(
                src_ref=out_ref.at[rows, :],
                dst_ref=out_ref.at[rows, :],
                send_sem=xs_send.at[c],
                recv_sem=xs_recv.at[c],
                device_id=x_peer,
                device_id_type=pl.DeviceIdType.MESH,
            )
            f.start()
            x_sends.append(f)

        keep = pltpu.make_async_remote_copy(
            src_ref=out_ref.at[pl.ds(opp_base + keep_lo, keep_sz), :],
            dst_ref=out_ref.at[pl.ds(opp_base + keep_lo, keep_sz), :],
            send_sem=ys_send.at[N_FWD_CHUNKS],
            recv_sem=ys_recv.at[N_FWD_CHUNKS],
            device_id=y_peer,
            device_id_type=pl.DeviceIdType.MESH,
        )
        keep.wait_recv()

        for c in range(N_FWD_CHUNKS):
            off = jnp.where(my_x == 0, DY_ROWS + c * fc, c * fc)
            rows = pl.ds(opp_base + off, fc)
            r = pltpu.make_async_remote_copy(
                src_ref=out_ref.at[rows, :],
                dst_ref=out_ref.at[rows, :],
                send_sem=xs_send.at[c],
                recv_sem=xs_recv.at[c],
                device_id=x_peer,
                device_id_type=pl.DeviceIdType.MESH,
            )
            r.wait_recv()

        for s in y_sends:
            s.wait_send()
        for s in x_sends:
            s.wait_send()

    return pl.pallas_call(
        body,
        out_shape=jax.ShapeDtypeStruct((2 * m, n), jnp.bfloat16),
        in_specs=[pl.BlockSpec(memory_space=pltpu.VMEM)],
        out_specs=pl.BlockSpec(memory_space=pltpu.VMEM),
        scratch_shapes=[
            pltpu.SemaphoreType.DMA((N_FWD_CHUNKS + 1,)),
            pltpu.SemaphoreType.DMA((N_FWD_CHUNKS + 1,)),
            pltpu.SemaphoreType.DMA((N_FWD_CHUNKS,)),
            pltpu.SemaphoreType.DMA((N_FWD_CHUNKS,)),
        ],
        compiler_params=pltpu.CompilerParams(collective_id=0),
    )(x)


# device time: 11048 ns/iter; 1.0426x vs baseline; 1.0282x over previous
import jax
import jax.numpy as jnp
from jax import lax
from jax.experimental import pallas as pl
from jax.experimental.pallas import tpu as pltpu

DY_ROWS = 352
N_FWD_CHUNKS = 2


def kernel(x):
    m, n = x.shape
    fwd_rows = m - DY_ROWS
    fc = fwd_rows // N_FWD_CHUNKS

    def body(x_ref, out_ref, ys_send, ys_recv, xs_send, xs_recv):
        my_x = lax.axis_index("x")
        my_y = lax.axis_index("y")
        y_peer = (my_x, 1 - my_y)
        x_peer = (1 - my_x, my_y)

        own_base = my_y * m
        opp_base = (1 - my_y) * m

        barrier_sem = pltpu.get_barrier_semaphore()
        for peer in (y_peer, x_peer):
            pl.semaphore_signal(
                barrier_sem, inc=1, device_id=peer,
                device_id_type=pl.DeviceIdType.MESH,
            )
        pl.semaphore_wait(barrier_sem, 2)

        keep_lo = jnp.where(my_x == 0, fwd_rows, m - DY_ROWS)
        keep_sz = DY_ROWS - fwd_rows
        send_offs = []
        for c in range(N_FWD_CHUNKS):
            lo = c * fc
            hi = DY_ROWS + c * fc
            send_offs.append(jnp.where(my_x == 0, lo, hi))

        y_sends = []
        for c in range(N_FWD_CHUNKS):
            off = send_offs[c]
            out_ref[pl.ds(own_base + off, fc), :] = x_ref[
                pl.ds(off, fc), :
            ].astype(out_ref.dtype)
            s = pltpu.make_async_remote_copy(
                src_ref=out_ref.at[pl.ds(own_base + off, fc), :],
                dst_ref=out_ref.at[pl.ds(own_base + off, fc), :],
                send_sem=ys_send.at[c],
                recv_sem=ys_recv.at[c],
                device_id=y_peer,
                device_id_type=pl.DeviceIdType.MESH,
            )
            s.start()
            y_sends.append(s)
        if keep_sz > 0:
            out_ref[pl.ds(own_base + keep_lo, keep_sz), :] = x_ref[
                pl.ds(keep_lo, keep_sz), :
            ].astype(out_ref.dtype)
            s = pltpu.make_async_remote_copy(
                src_ref=out_ref.at[pl.ds(own_base + keep_lo, keep_sz), :],
                dst_ref=out_ref.at[pl.ds(own_base + keep_lo, keep_sz), :],
                send_sem=ys_send.at[N_FWD_CHUNKS],
                recv_sem=ys_recv.at[N_FWD_CHUNKS],
                device_id=y_peer,
                device_id_type=pl.DeviceIdType.MESH,
            )
            s.start()
            y_sends.append(s)

        rest_lo = jnp.where(my_x == 0, DY_ROWS, 0)
        out_ref[pl.ds(own_base + rest_lo, fwd_rows), :] = x_ref[
            pl.ds(rest_lo, fwd_rows), :
        ].astype(out_ref.dtype)

        x_sends = []
        for c in range(N_FWD_CHUNKS):
            off = send_offs[c]
            rows = pl.ds(opp_base + off, fc)
            r = pltpu.make_async_remote_copy(
                src_ref=out_ref.at[rows, :],
                dst_ref=out_ref.at[rows, :],
                send_sem=ys_send.at[c],
                recv_sem=ys_recv.at[c],
                device_id=y_peer,
                device_id_type=pl.DeviceIdType.MESH,
            )
            r.wait_recv()
            f = pltpu.make_async_remote_copy(
                src_ref=out_ref.at[rows, :],
                dst_ref=out_ref.at[rows, :],
                send_sem=xs_send.at[c],
                recv_sem=xs_recv.at[c],
                device_id=x_peer,
                device_id_type=pl.DeviceIdType.MESH,
            )
            f.start()
            x_sends.append(f)

        if keep_sz > 0:
            keep = pltpu.make_async_remote_copy(
                src_ref=out_ref.at[pl.ds(opp_base + keep_lo, keep_sz), :],
                dst_ref=out_ref.at[pl.ds(opp_base + keep_lo, keep_sz), :],
                send_sem=ys_send.at[N_FWD_CHUNKS],
                recv_sem=ys_recv.at[N_FWD_CHUNKS],
                device_id=y_peer,
                device_id_type=pl.DeviceIdType.MESH,
            )
            keep.wait_recv()

        for c in range(N_FWD_CHUNKS):
            off = jnp.where(my_x == 0, DY_ROWS + c * fc, c * fc)
            rows = pl.ds(opp_base + off, fc)
            r = pltpu.make_async_remote_copy(
                src_ref=out_ref.at[rows, :],
                dst_ref=out_ref.at[rows, :],
                send_sem=xs_send.at[c],
                recv_sem=xs_recv.at[c],
                device_id=x_peer,
                device_id_type=pl.DeviceIdType.MESH,
            )
            r.wait_recv()

        for s in y_sends:
            s.wait_send()
        for s in x_sends:
            s.wait_send()

    return pl.pallas_call(
        body,
        out_shape=jax.ShapeDtypeStruct((2 * m, n), jnp.bfloat16),
        in_specs=[pl.BlockSpec(memory_space=pltpu.VMEM)],
        out_specs=pl.BlockSpec(memory_space=pltpu.VMEM),
        scratch_shapes=[
            pltpu.SemaphoreType.DMA((N_FWD_CHUNKS + 1,)),
            pltpu.SemaphoreType.DMA((N_FWD_CHUNKS + 1,)),
            pltpu.SemaphoreType.DMA((N_FWD_CHUNKS,)),
            pltpu.SemaphoreType.DMA((N_FWD_CHUNKS,)),
        ],
        compiler_params=pltpu.CompilerParams(collective_id=0),
    )(x)
